# baseline (speedup 1.0000x reference)
import functools

import jax
import jax.numpy as jnp
from jax import lax
from jax.experimental import pallas as pl
from jax.experimental.pallas import tpu as pltpu
from jax.experimental.pallas import tpu_sc as plsc

N_NODES = 10000
N_EDGES = 320000
D_FEAT = 128

NC, NS = 2, 16
NPAD = 10112
DUMMY = N_NODES
KCH = 80
CHT = NC * NS * KCH
EPAD = CHT * 128
RPW = NPAD // NS
NIO = N_NODES
NB = 1000
GRID = NIO // NB
HV = 16

_MESH = dict(core_axis_name="c", subcore_axis_name="s")


def _zero_fill(buf, d):
    zvec = jnp.zeros((16,), jnp.float32)

    @pl.loop(0, 128)
    def _(r):
        for q in range(d // 16):
            buf[r, pl.ds(q * 16, 16)] = zvec


def _init_accum(src_buf, accum, base):
    for t in range(RPW // 128):
        pltpu.sync_copy(src_buf, accum.at[pl.ds(base + t * 128, 128)])
    rem = RPW % 128
    if rem:
        pltpu.sync_copy(src_buf.at[pl.ds(0, rem)],
                        accum.at[pl.ds(base + (RPW // 128) * 128, rem)])


def _make_sc_scatter(d):
    dh = d // 2
    kc = CHT // NS
    nbuf = 4 if dh >= 64 else 8

    @functools.partial(
        pl.kernel,
        out_type=jax.ShapeDtypeStruct((NC, NPAD, dh), jnp.float32),
        mesh=plsc.VectorSubcoreMesh(**_MESH),
        compiler_params=pltpu.CompilerParams(use_tc_tiling_on_sc=False),
        scratch_types=[
            pltpu.VMEM((kc, 128), jnp.int32),
            pltpu.VMEM((kc, 128), jnp.int32),
        ] + [pltpu.VMEM((128, dh), jnp.float32) for _ in range(nbuf)] + [
            pltpu.VMEM_SHARED((NPAD, dh), jnp.float32),
        ] + [pltpu.SemaphoreType.DMA for _ in range(2 * nbuf + 2)],
    )
    def sc_scatter(y_hbm, src_hbm, dst_hbm, out_hbm, src_v, dst_v, *rest):
        rows = rest[:nbuf]
        accum = rest[nbuf]
        gsem = rest[nbuf + 1:nbuf + 1 + nbuf]
        ssem = rest[nbuf + 1 + nbuf:nbuf + 1 + 2 * nbuf]
        isem0, isem1 = rest[nbuf + 1 + 2 * nbuf:]
        c = lax.axis_index("c")
        s = lax.axis_index("s")
        yc = y_hbm.at[c]

        icp0 = pltpu.async_copy(src_hbm.at[pl.ds(s * kc, kc)], src_v, isem0)
        icp1 = pltpu.async_copy(dst_hbm.at[pl.ds(s * kc, kc)], dst_v, isem1)

        base = s * RPW
        _zero_fill(rows[0], dh)
        _init_accum(rows[0], accum, base)
        icp0.wait()
        icp1.wait()
        plsc.subcore_barrier()

        for b in range(nbuf):
            pltpu.async_copy(yc.at[src_v.at[b]], rows[b], gsem[b])

        @pl.loop(0, kc, step=nbuf)
        def _(j):
            for b in range(nbuf):
                pltpu.make_async_copy(yc.at[src_v.at[0]], rows[b],
                                      gsem[b]).wait()
                pltpu.async_copy(rows[b], accum.at[dst_v.at[j + b]],
                                 ssem[b], add=True)
            for b in range(nbuf):
                pltpu.make_async_copy(rows[b], accum.at[dst_v.at[0]],
                                      ssem[b]).wait()

                @pl.when(j + b + nbuf < kc)
                def _():
                    pltpu.async_copy(yc.at[src_v.at[j + b + nbuf]],
                                     rows[b], gsem[b])

        plsc.subcore_barrier()
        pltpu.sync_copy(accum.at[pl.ds(base, RPW)],
                        out_hbm.at[c].at[pl.ds(base, RPW)])

    return sc_scatter


def _make_sc_hist():
    @functools.partial(
        pl.kernel,
        out_type=jax.ShapeDtypeStruct((NC, NPAD, HV), jnp.float32),
        mesh=plsc.VectorSubcoreMesh(**_MESH),
        compiler_params=pltpu.CompilerParams(use_tc_tiling_on_sc=False),
        scratch_types=[
            pltpu.VMEM((KCH, 128), jnp.int32),
            pltpu.VMEM((128, HV), jnp.float32),
            pltpu.VMEM_SHARED((NPAD, HV), jnp.float32),
        ],
    )
    def sc_hist(dst_hbm, out_hbm, dst_v, vals, accum):
        c = lax.axis_index("c")
        s = lax.axis_index("s")
        w = c * NS + s

        pltpu.sync_copy(dst_hbm.at[pl.ds(w * KCH, KCH)], dst_v)

        base = s * RPW
        _zero_fill(vals, HV)
        _init_accum(vals, accum, base)
        plsc.subcore_barrier()

        ovec = jnp.ones((16,), jnp.float32)

        @pl.loop(0, 128)
        def _(r):
            vals[r, pl.ds(0, 16)] = ovec

        @pl.loop(0, KCH)
        def _(j):
            pltpu.sync_copy(vals, accum.at[dst_v.at[j]], add=True)

        plsc.subcore_barrier()
        pltpu.sync_copy(accum.at[pl.ds(base, RPW)],
                        out_hbm.at[c].at[pl.ds(base, RPW)])

    return sc_hist


_SC_HIST = _make_sc_hist()
_SC_SCATTER = {d: _make_sc_scatter(d) for d in (128, 64, 32)}


def _node_spec(d):
    return pl.BlockSpec((NB, d), lambda i: (i, 0))


def _half_spec(dh):
    return pl.BlockSpec((2, NB, dh), lambda i: (0, i, 0))


def _full_spec(shape):
    return pl.BlockSpec(shape, lambda i: (0, 0))


def _dis(hp0_ref, hp1_ref):
    deg = hp0_ref[...][:, :1] + hp1_ref[...][:, :1] + 1.0
    return lax.rsqrt(deg)


def _dot(a, w_ref):
    return jnp.dot(a, w_ref[...], preferred_element_type=jnp.float32,
                   precision=lax.Precision.HIGHEST)


def _first_body(hp0_ref, hp1_ref, x_ref, w_ref, o_ref):
    y = _dis(hp0_ref, hp1_ref) * _dot(x_ref[...], w_ref)
    dh = o_ref.shape[2]
    o_ref[...] = jnp.stack([y[:, :dh], y[:, dh:]], axis=0)


def _combined(hp0_ref, hp1_ref, a_ref, y_ref, b_ref):
    a = a_ref[...] + y_ref[...]
    full = jnp.concatenate([a[0], a[1]], axis=1)
    p = _dis(hp0_ref, hp1_ref) * full
    return jnp.maximum(p + b_ref[...][:1, :], 0.0)


def _halves(y):
    dh = y.shape[1] // 2
    return jnp.stack([y[:, :dh], y[:, dh:]], axis=0)


def _make_combine_body(with_matmul):
    if with_matmul:
        def body(hp0_ref, hp1_ref, a_ref, y_ref, b_ref, w_ref, o_ref):
            h = _combined(hp0_ref, hp1_ref, a_ref, y_ref, b_ref)
            o_ref[...] = _halves(_dis(hp0_ref, hp1_ref) * _dot(h, w_ref))
    else:
        def body(hp0_ref, hp1_ref, a_ref, y_ref, b_ref, o_ref):
            h = _combined(hp0_ref, hp1_ref, a_ref, y_ref, b_ref)
            o_ref[...] = _halves(_dis(hp0_ref, hp1_ref) * h)
    return body


def _final_body(hp0_ref, hp1_ref, a_ref, y_ref, b_ref, w_ref, o_ref):
    a = a_ref[...] + y_ref[...]
    full = jnp.concatenate([a[0], a[1]], axis=1)
    p = _dis(hp0_ref, hp1_ref) * full
    o_ref[...] = _dot(p, w_ref) + b_ref[...][:1, :]


_HP_SPECS = [_node_spec(HV), _node_spec(HV)]


def _combine_specs(d, dn, with_w, db):
    dh = d // 2
    specs = _HP_SPECS + [_half_spec(dh), _half_spec(dh), _full_spec((8, db))]
    if with_w:
        specs.append(_full_spec((d, dn)))
    return specs


def _tc_call(body, in_specs, out_shape, out_spec, name):
    return pl.pallas_call(
        body,
        grid=(GRID,),
        in_specs=in_specs,
        out_specs=out_spec,
        out_shape=jax.ShapeDtypeStruct(out_shape, jnp.float32),
        name=name,
    )


def kernel(x, edge_index, W1, b1, W2, b2, W3, b3, W4, b4):
    ei = edge_index.astype(jnp.int32)
    src2 = jnp.concatenate(
        [ei[0], jnp.zeros((EPAD - N_EDGES,), jnp.int32)]).reshape(CHT, 128)
    dst2 = jnp.concatenate(
        [ei[1], jnp.full((EPAD - N_EDGES,), DUMMY, jnp.int32)]).reshape(CHT, 128)
    bb1 = jnp.broadcast_to(b1, (8, b1.shape[0]))
    bb2 = jnp.broadcast_to(b2, (8, b2.shape[0]))
    bb3 = jnp.broadcast_to(b3, (8, b3.shape[0]))
    bb4 = jnp.broadcast_to(b4, (8, b4.shape[0]))

    hp = _SC_HIST(dst2)
    hp0, hp1 = hp[0], hp[1]

    y1 = _tc_call(_first_body,
                  _HP_SPECS + [_node_spec(128), _full_spec((128, 128))],
                  (2, NIO, 64), _half_spec(64),
                  "gcn_first")(hp0, hp1, x, W1)
    g1 = _SC_SCATTER[128](y1, src2, dst2)
    y2 = _tc_call(_make_combine_body(True), _combine_specs(128, 64, True, 128),
                  (2, NIO, 32), _half_spec(32),
                  "gcn_comb1")(hp0, hp1, g1, y1, bb1, W2)
    g2 = _SC_SCATTER[64](y2, src2, dst2)
    y3 = _tc_call(_make_combine_body(True), _combine_specs(64, 32, True, 64),
                  (2, NIO, 16), _half_spec(16),
                  "gcn_comb2")(hp0, hp1, g2, y2, bb2, W3)
    g3 = _SC_SCATTER[32](y3, src2, dst2)
    y4 = _tc_call(_make_combine_body(False), _combine_specs(32, 0, False, 32),
                  (2, NIO, 16), _half_spec(16),
                  "gcn_comb3")(hp0, hp1, g3, y3, bb3)
    g4 = _SC_SCATTER[32](y4, src2, dst2)
    out = _tc_call(_final_body, _combine_specs(32, 40, True, 40),
                   (NIO, 40), _node_spec(40),
                   "gcn_final")(hp0, hp1, g4, y4, bb4, W4)
    return out


# speedup vs baseline: 1.0453x; 1.0453x over previous
import functools

import jax
import jax.numpy as jnp
from jax import lax
from jax.experimental import pallas as pl
from jax.experimental.pallas import tpu as pltpu
from jax.experimental.pallas import tpu_sc as plsc

N_NODES = 10000
N_EDGES = 320000
D_FEAT = 128

NC, NS = 2, 16
NPAD = 10112
DUMMY = N_NODES
KCH = 80
CHT = NC * NS * KCH
EPAD = CHT * 128
RPW = NPAD // NS
NB = 1264
GRID = NPAD // NB
HV = 16

_MESH = dict(core_axis_name="c", subcore_axis_name="s")


def _zero_fill(buf, d):
    zvec = jnp.zeros((16,), jnp.float32)

    @pl.loop(0, 128)
    def _(r):
        for q in range(d // 16):
            buf[r, pl.ds(q * 16, 16)] = zvec


def _init_accum(src_buf, accum, base):
    for t in range(RPW // 128):
        pltpu.sync_copy(src_buf, accum.at[pl.ds(base + t * 128, 128)])
    rem = RPW % 128
    if rem:
        pltpu.sync_copy(src_buf.at[pl.ds(0, rem)],
                        accum.at[pl.ds(base + (RPW // 128) * 128, rem)])


def _make_sc_scatter(d):
    dh = d // 2
    kc = CHT // NS
    nbuf = 4 if dh >= 64 else 8

    @functools.partial(
        pl.kernel,
        out_type=jax.ShapeDtypeStruct((NC, NPAD, dh), jnp.float32),
        mesh=plsc.VectorSubcoreMesh(**_MESH),
        compiler_params=pltpu.CompilerParams(use_tc_tiling_on_sc=False),
        scratch_types=[
            pltpu.VMEM((kc, 128), jnp.int32),
            pltpu.VMEM((kc, 128), jnp.int32),
        ] + [pltpu.VMEM((128, dh), jnp.float32) for _ in range(nbuf)] + [
            pltpu.VMEM_SHARED((NPAD, dh), jnp.float32),
        ] + [pltpu.SemaphoreType.DMA for _ in range(2 * nbuf + 2)],
    )
    def sc_scatter(y_hbm, src_hbm, dst_hbm, out_hbm, src_v, dst_v, *rest):
        rows = rest[:nbuf]
        accum = rest[nbuf]
        gsem = rest[nbuf + 1:nbuf + 1 + nbuf]
        ssem = rest[nbuf + 1 + nbuf:nbuf + 1 + 2 * nbuf]
        isem0, isem1 = rest[nbuf + 1 + 2 * nbuf:]
        c = lax.axis_index("c")
        s = lax.axis_index("s")
        yc = y_hbm.at[c]

        icp0 = pltpu.async_copy(src_hbm.at[pl.ds(s * kc, kc)], src_v, isem0)
        icp1 = pltpu.async_copy(dst_hbm.at[pl.ds(s * kc, kc)], dst_v, isem1)

        base = s * RPW
        _zero_fill(rows[0], dh)
        _init_accum(rows[0], accum, base)
        icp0.wait()
        icp1.wait()
        plsc.subcore_barrier()

        for b in range(nbuf):
            pltpu.async_copy(yc.at[src_v.at[b]], rows[b], gsem[b])

        @pl.loop(0, kc, step=nbuf)
        def _(j):
            for b in range(nbuf):
                pltpu.make_async_copy(yc.at[src_v.at[0]], rows[b],
                                      gsem[b]).wait()
                pltpu.async_copy(rows[b], accum.at[dst_v.at[j + b]],
                                 ssem[b], add=True)
            for b in range(nbuf):
                pltpu.make_async_copy(rows[b], accum.at[dst_v.at[0]],
                                      ssem[b]).wait()

                @pl.when(j + b + nbuf < kc)
                def _():
                    pltpu.async_copy(yc.at[src_v.at[j + b + nbuf]],
                                     rows[b], gsem[b])

        plsc.subcore_barrier()
        pltpu.sync_copy(accum.at[pl.ds(base, RPW)],
                        out_hbm.at[c].at[pl.ds(base, RPW)])

    return sc_scatter


def _make_sc_hist():
    @functools.partial(
        pl.kernel,
        out_type=jax.ShapeDtypeStruct((NC, NPAD, HV), jnp.float32),
        mesh=plsc.VectorSubcoreMesh(**_MESH),
        compiler_params=pltpu.CompilerParams(use_tc_tiling_on_sc=False),
        scratch_types=[
            pltpu.VMEM((KCH, 128), jnp.int32),
            pltpu.VMEM((128, HV), jnp.float32),
            pltpu.VMEM_SHARED((NPAD, HV), jnp.float32),
        ],
    )
    def sc_hist(dst_hbm, out_hbm, dst_v, vals, accum):
        c = lax.axis_index("c")
        s = lax.axis_index("s")
        w = c * NS + s

        pltpu.sync_copy(dst_hbm.at[pl.ds(w * KCH, KCH)], dst_v)

        base = s * RPW
        _zero_fill(vals, HV)
        _init_accum(vals, accum, base)
        plsc.subcore_barrier()

        ovec = jnp.ones((16,), jnp.float32)

        @pl.loop(0, 128)
        def _(r):
            vals[r, pl.ds(0, 16)] = ovec

        @pl.loop(0, KCH)
        def _(j):
            pltpu.sync_copy(vals, accum.at[dst_v.at[j]], add=True)

        plsc.subcore_barrier()
        pltpu.sync_copy(accum.at[pl.ds(base, RPW)],
                        out_hbm.at[c].at[pl.ds(base, RPW)])

    return sc_hist


_SC_HIST = _make_sc_hist()
_SC_SCATTER = {d: _make_sc_scatter(d) for d in (128, 64, 32)}


def _node_spec(d):
    return pl.BlockSpec((NB, d), lambda i: (i, 0))


def _half_spec(dh):
    return pl.BlockSpec((2, NB, dh), lambda i: (0, i, 0))


def _full_spec(shape):
    return pl.BlockSpec(shape, lambda i: (0, 0))


def _dis(hp0_ref, hp1_ref):
    deg = hp0_ref[...][:, :1] + hp1_ref[...][:, :1] + 1.0
    return lax.rsqrt(deg)


def _dot(a, w_ref):
    return jnp.dot(a, w_ref[...], preferred_element_type=jnp.float32,
                   precision=lax.Precision.HIGHEST)


def _first_body(hp0_ref, hp1_ref, x_ref, w_ref, o_ref):
    y = _dis(hp0_ref, hp1_ref) * _dot(x_ref[...], w_ref)
    dh = o_ref.shape[2]
    o_ref[...] = jnp.stack([y[:, :dh], y[:, dh:]], axis=0)


def _combined(hp0_ref, hp1_ref, a_ref, y_ref, b_ref):
    a = a_ref[...] + y_ref[...]
    full = jnp.concatenate([a[0], a[1]], axis=1)
    p = _dis(hp0_ref, hp1_ref) * full
    return jnp.maximum(p + b_ref[...][:1, :], 0.0)


def _halves(y):
    dh = y.shape[1] // 2
    return jnp.stack([y[:, :dh], y[:, dh:]], axis=0)


def _make_combine_body(with_matmul):
    if with_matmul:
        def body(hp0_ref, hp1_ref, a_ref, y_ref, b_ref, w_ref, o_ref):
            h = _combined(hp0_ref, hp1_ref, a_ref, y_ref, b_ref)
            o_ref[...] = _halves(_dis(hp0_ref, hp1_ref) * _dot(h, w_ref))
    else:
        def body(hp0_ref, hp1_ref, a_ref, y_ref, b_ref, o_ref):
            h = _combined(hp0_ref, hp1_ref, a_ref, y_ref, b_ref)
            o_ref[...] = _halves(_dis(hp0_ref, hp1_ref) * h)
    return body


def _final_body(hp0_ref, hp1_ref, a_ref, y_ref, b_ref, w_ref, o_ref):
    a = a_ref[...] + y_ref[...]
    full = jnp.concatenate([a[0], a[1]], axis=1)
    p = _dis(hp0_ref, hp1_ref) * full
    o_ref[...] = _dot(p, w_ref) + b_ref[...][:1, :]


_HP_SPECS = [_node_spec(HV), _node_spec(HV)]


def _combine_specs(d, dn, with_w, db):
    dh = d // 2
    specs = _HP_SPECS + [_half_spec(dh), _half_spec(dh), _full_spec((8, db))]
    if with_w:
        specs.append(_full_spec((d, dn)))
    return specs


def _tc_call(body, in_specs, out_shape, out_spec, name):
    return pl.pallas_call(
        body,
        grid=(GRID,),
        in_specs=in_specs,
        out_specs=out_spec,
        out_shape=jax.ShapeDtypeStruct(out_shape, jnp.float32),
        name=name,
    )


def kernel(x, edge_index, W1, b1, W2, b2, W3, b3, W4, b4):
    xp = jnp.concatenate(
        [x, jnp.zeros((NPAD - N_NODES, D_FEAT), jnp.float32)], axis=0)
    ei = edge_index.astype(jnp.int32)
    src2 = jnp.concatenate(
        [ei[0], jnp.zeros((EPAD - N_EDGES,), jnp.int32)]).reshape(CHT, 128)
    dst2 = jnp.concatenate(
        [ei[1], jnp.full((EPAD - N_EDGES,), DUMMY, jnp.int32)]).reshape(CHT, 128)
    bb1 = jnp.broadcast_to(b1, (8, b1.shape[0]))
    bb2 = jnp.broadcast_to(b2, (8, b2.shape[0]))
    bb3 = jnp.broadcast_to(b3, (8, b3.shape[0]))
    bb4 = jnp.broadcast_to(b4, (8, b4.shape[0]))

    hp = _SC_HIST(dst2)
    hp0, hp1 = hp[0], hp[1]

    y1 = _tc_call(_first_body,
                  _HP_SPECS + [_node_spec(128), _full_spec((128, 128))],
                  (2, NPAD, 64), _half_spec(64),
                  "gcn_first")(hp0, hp1, xp, W1)
    g1 = _SC_SCATTER[128](y1, src2, dst2)
    y2 = _tc_call(_make_combine_body(True), _combine_specs(128, 64, True, 128),
                  (2, NPAD, 32), _half_spec(32),
                  "gcn_comb1")(hp0, hp1, g1, y1, bb1, W2)
    g2 = _SC_SCATTER[64](y2, src2, dst2)
    y3 = _tc_call(_make_combine_body(True), _combine_specs(64, 32, True, 64),
                  (2, NPAD, 16), _half_spec(16),
                  "gcn_comb2")(hp0, hp1, g2, y2, bb2, W3)
    g3 = _SC_SCATTER[32](y3, src2, dst2)
    y4 = _tc_call(_make_combine_body(False), _combine_specs(32, 0, False, 32),
                  (2, NPAD, 16), _half_spec(16),
                  "gcn_comb3")(hp0, hp1, g3, y3, bb3)
    g4 = _SC_SCATTER[32](y4, src2, dst2)
    out = _tc_call(_final_body, _combine_specs(32, 40, True, 40),
                   (NPAD, 40), _node_spec(40),
                   "gcn_final")(hp0, hp1, g4, y4, bb4, W4)
    return out[:N_NODES]


# speedup vs baseline: 1.9508x; 1.8663x over previous
import functools

import jax
import jax.numpy as jnp
from jax import lax
from jax.experimental import pallas as pl
from jax.experimental.pallas import tpu as pltpu
from jax.experimental.pallas import tpu_sc as plsc

N_NODES = 10000
N_EDGES = 320000
D_FEAT = 128

NC, NS = 2, 16
NPAD = 10112
CHT = N_EDGES // 128
KBASE = CHT // NS
KEXTRA = CHT % NS
KMAX = KBASE + 1
HBASE = (CHT // 2) // NS
HEXTRA = (CHT // 2) % NS
HMAX = HBASE + 1
RPW = NPAD // NS
NB = 1264
GRID = NPAD // NB
HV = 16

_MESH = dict(core_axis_name="c", subcore_axis_name="s")


def _zero_fill(buf, d):
    zvec = jnp.zeros((16,), jnp.float32)

    @pl.loop(0, 128)
    def _(r):
        for q in range(d // 16):
            buf[r, pl.ds(q * 16, 16)] = zvec


def _init_accum(src_buf, accum, base):
    for t in range(RPW // 128):
        pltpu.sync_copy(src_buf, accum.at[pl.ds(base + t * 128, 128)])
    rem = RPW % 128
    if rem:
        pltpu.sync_copy(src_buf.at[pl.ds(0, rem)],
                        accum.at[pl.ds(base + (RPW // 128) * 128, rem)])


def _make_sc_scatter(d):
    dh = d // 2
    nbuf = 4 if dh >= 64 else 8

    @functools.partial(
        pl.kernel,
        out_type=jax.ShapeDtypeStruct((NC, NPAD, dh), jnp.float32),
        mesh=plsc.VectorSubcoreMesh(**_MESH),
        compiler_params=pltpu.CompilerParams(use_tc_tiling_on_sc=False),
        scratch_types=[
            pltpu.VMEM((KMAX, 128), jnp.int32),
            pltpu.VMEM((KMAX, 128), jnp.int32),
        ] + [pltpu.VMEM((128, dh), jnp.float32) for _ in range(nbuf)] + [
            pltpu.VMEM_SHARED((NPAD, dh), jnp.float32),
        ] + [pltpu.SemaphoreType.DMA for _ in range(2 * nbuf + 2)],
    )
    def sc_scatter(y_hbm, src_hbm, dst_hbm, out_hbm, src_v, dst_v, *rest):
        rows = rest[:nbuf]
        accum = rest[nbuf]
        gsem = rest[nbuf + 1:nbuf + 1 + nbuf]
        ssem = rest[nbuf + 1 + nbuf:nbuf + 1 + 2 * nbuf]
        isem0, isem1 = rest[nbuf + 1 + 2 * nbuf:]
        c = lax.axis_index("c")
        s = lax.axis_index("s")
        yc = y_hbm.at[c]
        start = s * KBASE + jnp.minimum(s, KEXTRA)
        kc = jnp.where(s < KEXTRA, KMAX, KBASE)
        kfloor = (kc // nbuf) * nbuf

        @pl.when(s < KEXTRA)
        def _():
            pltpu.async_copy(src_hbm.at[pl.ds(start, KMAX)], src_v, isem0)
            pltpu.async_copy(dst_hbm.at[pl.ds(start, KMAX)], dst_v, isem1)

        @pl.when(s >= KEXTRA)
        def _():
            pltpu.async_copy(src_hbm.at[pl.ds(start, KBASE)],
                             src_v.at[pl.ds(0, KBASE)], isem0)
            pltpu.async_copy(dst_hbm.at[pl.ds(start, KBASE)],
                             dst_v.at[pl.ds(0, KBASE)], isem1)

        base = s * RPW
        _zero_fill(rows[0], dh)
        _init_accum(rows[0], accum, base)

        @pl.when(s < KEXTRA)
        def _():
            pltpu.make_async_copy(src_hbm.at[pl.ds(0, KMAX)], src_v,
                                  isem0).wait()
            pltpu.make_async_copy(dst_hbm.at[pl.ds(0, KMAX)], dst_v,
                                  isem1).wait()

        @pl.when(s >= KEXTRA)
        def _():
            pltpu.make_async_copy(src_hbm.at[pl.ds(0, KBASE)],
                                  src_v.at[pl.ds(0, KBASE)], isem0).wait()
            pltpu.make_async_copy(dst_hbm.at[pl.ds(0, KBASE)],
                                  dst_v.at[pl.ds(0, KBASE)], isem1).wait()
        plsc.subcore_barrier()

        for b in range(nbuf):
            pltpu.async_copy(yc.at[src_v.at[b]], rows[b], gsem[b])

        @pl.loop(0, kfloor, step=nbuf)
        def _(j):
            for b in range(nbuf):
                pltpu.make_async_copy(yc.at[src_v.at[0]], rows[b],
                                      gsem[b]).wait()
                pltpu.async_copy(rows[b], accum.at[dst_v.at[j + b]],
                                 ssem[b], add=True)
            for b in range(nbuf):
                pltpu.make_async_copy(rows[b], accum.at[dst_v.at[0]],
                                      ssem[b]).wait()

                @pl.when(j + b + nbuf < kc)
                def _():
                    pltpu.async_copy(yc.at[src_v.at[j + b + nbuf]],
                                     rows[b], gsem[b])

        for b in range(nbuf - 1):
            @pl.when(kfloor + b < kc)
            def _():
                pltpu.make_async_copy(yc.at[src_v.at[0]], rows[b],
                                      gsem[b]).wait()
                pltpu.async_copy(rows[b], accum.at[dst_v.at[kfloor + b]],
                                 ssem[b], add=True)
                pltpu.make_async_copy(rows[b], accum.at[dst_v.at[0]],
                                      ssem[b]).wait()

        plsc.subcore_barrier()
        pltpu.sync_copy(accum.at[pl.ds(base, RPW)],
                        out_hbm.at[c].at[pl.ds(base, RPW)])

    return sc_scatter


def _make_sc_hist():
    @functools.partial(
        pl.kernel,
        out_type=jax.ShapeDtypeStruct((NC, NPAD, HV), jnp.float32),
        mesh=plsc.VectorSubcoreMesh(**_MESH),
        compiler_params=pltpu.CompilerParams(use_tc_tiling_on_sc=False),
        scratch_types=[
            pltpu.VMEM((HMAX, 128), jnp.int32),
            pltpu.VMEM((128, HV), jnp.float32),
            pltpu.VMEM_SHARED((NPAD, HV), jnp.float32),
        ],
    )
    def sc_hist(dst_hbm, out_hbm, dst_v, vals, accum):
        c = lax.axis_index("c")
        s = lax.axis_index("s")
        w = c * NS + s
        start = w * HBASE + jnp.minimum(w, 2 * HEXTRA)
        kc = jnp.where(w < 2 * HEXTRA, HMAX, HBASE)

        @pl.when(w < 2 * HEXTRA)
        def _():
            pltpu.sync_copy(dst_hbm.at[pl.ds(start, HMAX)], dst_v)

        @pl.when(w >= 2 * HEXTRA)
        def _():
            pltpu.sync_copy(dst_hbm.at[pl.ds(start, HBASE)],
                            dst_v.at[pl.ds(0, HBASE)])

        base = s * RPW
        _zero_fill(vals, HV)
        _init_accum(vals, accum, base)
        plsc.subcore_barrier()

        ovec = jnp.ones((16,), jnp.float32)

        @pl.loop(0, 128)
        def _(r):
            vals[r, pl.ds(0, 16)] = ovec

        @pl.loop(0, kc)
        def _(j):
            pltpu.sync_copy(vals, accum.at[dst_v.at[j]], add=True)

        plsc.subcore_barrier()
        pltpu.sync_copy(accum.at[pl.ds(base, RPW)],
                        out_hbm.at[c].at[pl.ds(base, RPW)])

    return sc_hist


_SC_HIST = _make_sc_hist()
_SC_SCATTER = {d: _make_sc_scatter(d) for d in (128, 64, 32)}


def _node_spec(d):
    return pl.BlockSpec((NB, d), lambda i: (i, 0))


def _half_spec(dh):
    return pl.BlockSpec((2, NB, dh), lambda i: (0, i, 0))


def _full_spec(shape):
    return pl.BlockSpec(shape, lambda i: (0, 0))


def _dis(hp0_ref, hp1_ref):
    deg = hp0_ref[...][:, :1] + hp1_ref[...][:, :1] + 1.0
    return lax.rsqrt(deg)


def _dot(a, w_ref):
    return jnp.dot(a, w_ref[...], preferred_element_type=jnp.float32,
                   precision=lax.Precision.HIGHEST)


def _first_body(hp0_ref, hp1_ref, x_ref, w_ref, o_ref):
    y = _dis(hp0_ref, hp1_ref) * _dot(x_ref[...], w_ref)
    dh = o_ref.shape[2]
    o_ref[...] = jnp.stack([y[:, :dh], y[:, dh:]], axis=0)


def _combined(hp0_ref, hp1_ref, a_ref, y_ref, b_ref):
    a = a_ref[...] + y_ref[...]
    full = jnp.concatenate([a[0], a[1]], axis=1)
    p = _dis(hp0_ref, hp1_ref) * full
    return jnp.maximum(p + b_ref[...][:1, :], 0.0)


def _halves(y):
    dh = y.shape[1] // 2
    return jnp.stack([y[:, :dh], y[:, dh:]], axis=0)


def _make_combine_body(with_matmul):
    if with_matmul:
        def body(hp0_ref, hp1_ref, a_ref, y_ref, b_ref, w_ref, o_ref):
            h = _combined(hp0_ref, hp1_ref, a_ref, y_ref, b_ref)
            o_ref[...] = _halves(_dis(hp0_ref, hp1_ref) * _dot(h, w_ref))
    else:
        def body(hp0_ref, hp1_ref, a_ref, y_ref, b_ref, o_ref):
            h = _combined(hp0_ref, hp1_ref, a_ref, y_ref, b_ref)
            o_ref[...] = _halves(_dis(hp0_ref, hp1_ref) * h)
    return body


def _final_body(hp0_ref, hp1_ref, a_ref, y_ref, b_ref, w_ref, o_ref):
    a = a_ref[...] + y_ref[...]
    full = jnp.concatenate([a[0], a[1]], axis=1)
    p = _dis(hp0_ref, hp1_ref) * full
    o_ref[...] = _dot(p, w_ref) + b_ref[...][:1, :]


_HP_SPECS = [_node_spec(HV), _node_spec(HV)]


def _combine_specs(d, dn, with_w, db):
    dh = d // 2
    specs = _HP_SPECS + [_half_spec(dh), _half_spec(dh), _full_spec((8, db))]
    if with_w:
        specs.append(_full_spec((d, dn)))
    return specs


def _tc_call(body, in_specs, out_shape, out_spec, name):
    return pl.pallas_call(
        body,
        grid=(GRID,),
        in_specs=in_specs,
        out_specs=out_spec,
        out_shape=jax.ShapeDtypeStruct(out_shape, jnp.float32),
        name=name,
    )


def kernel(x, edge_index, W1, b1, W2, b2, W3, b3, W4, b4):
    xp = jnp.concatenate(
        [x, jnp.zeros((NPAD - N_NODES, D_FEAT), jnp.float32)], axis=0)
    ei = edge_index.astype(jnp.int32)
    src2 = ei[0].reshape(CHT, 128)
    dst2 = ei[1].reshape(CHT, 128)
    bb1 = jnp.broadcast_to(b1, (8, b1.shape[0]))
    bb2 = jnp.broadcast_to(b2, (8, b2.shape[0]))
    bb3 = jnp.broadcast_to(b3, (8, b3.shape[0]))
    bb4 = jnp.broadcast_to(b4, (8, b4.shape[0]))

    hp = _SC_HIST(dst2)
    hp0, hp1 = hp[0], hp[1]

    y1 = _tc_call(_first_body,
                  _HP_SPECS + [_node_spec(128), _full_spec((128, 128))],
                  (2, NPAD, 64), _half_spec(64),
                  "gcn_first")(hp0, hp1, xp, W1)
    g1 = _SC_SCATTER[128](y1, src2, dst2)
    y2 = _tc_call(_make_combine_body(True), _combine_specs(128, 64, True, 128),
                  (2, NPAD, 32), _half_spec(32),
                  "gcn_comb1")(hp0, hp1, g1, y1, bb1, W2)
    g2 = _SC_SCATTER[64](y2, src2, dst2)
    y3 = _tc_call(_make_combine_body(True), _combine_specs(64, 32, True, 64),
                  (2, NPAD, 16), _half_spec(16),
                  "gcn_comb2")(hp0, hp1, g2, y2, bb2, W3)
    g3 = _SC_SCATTER[32](y3, src2, dst2)
    y4 = _tc_call(_make_combine_body(False), _combine_specs(32, 0, False, 32),
                  (2, NPAD, 16), _half_spec(16),
                  "gcn_comb3")(hp0, hp1, g3, y3, bb3)
    g4 = _SC_SCATTER[32](y4, src2, dst2)
    out = _tc_call(_final_body, _combine_specs(32, 40, True, 40),
                   (NPAD, 40), _node_spec(40),
                   "gcn_final")(hp0, hp1, g4, y4, bb4, W4)
    return out[:N_NODES]


# speedup vs baseline: 1.9976x; 1.0240x over previous
import functools

import jax
import jax.numpy as jnp
from jax import lax
from jax.experimental import pallas as pl
from jax.experimental.pallas import tpu as pltpu
from jax.experimental.pallas import tpu_sc as plsc

N_NODES = 10000
N_EDGES = 320000
D_FEAT = 128

NC, NS = 2, 16
NPAD = 10112
CHT = N_EDGES // 128
KBASE = CHT // NS
KEXTRA = CHT % NS
KMAX = KBASE + 1
HBASE = (CHT // 2) // NS
HEXTRA = (CHT // 2) % NS
HMAX = HBASE + 1
RPW = NPAD // NS
NB = 2528
GRID = NPAD // NB
HV = 16

_MESH = dict(core_axis_name="c", subcore_axis_name="s")


def _zero_fill(buf, d):
    zvec = jnp.zeros((16,), jnp.float32)

    @pl.loop(0, 128)
    def _(r):
        for q in range(d // 16):
            buf[r, pl.ds(q * 16, 16)] = zvec


def _init_accum(src_buf, accum, base):
    for t in range(RPW // 128):
        pltpu.sync_copy(src_buf, accum.at[pl.ds(base + t * 128, 128)])
    rem = RPW % 128
    if rem:
        pltpu.sync_copy(src_buf.at[pl.ds(0, rem)],
                        accum.at[pl.ds(base + (RPW // 128) * 128, rem)])


def _make_sc_scatter(d):
    dh = d // 2
    nbuf = 5 if dh >= 64 else 8

    @functools.partial(
        pl.kernel,
        out_type=jax.ShapeDtypeStruct((NC, NPAD, dh), jnp.float32),
        mesh=plsc.VectorSubcoreMesh(**_MESH),
        compiler_params=pltpu.CompilerParams(use_tc_tiling_on_sc=False),
        scratch_types=[
            pltpu.VMEM((KMAX, 128), jnp.int32),
            pltpu.VMEM((KMAX, 128), jnp.int32),
        ] + [pltpu.VMEM((128, dh), jnp.float32) for _ in range(nbuf)] + [
            pltpu.VMEM_SHARED((NPAD, dh), jnp.float32),
        ] + [pltpu.SemaphoreType.DMA for _ in range(2 * nbuf + 2)],
    )
    def sc_scatter(y_hbm, src_hbm, dst_hbm, out_hbm, src_v, dst_v, *rest):
        rows = rest[:nbuf]
        accum = rest[nbuf]
        gsem = rest[nbuf + 1:nbuf + 1 + nbuf]
        ssem = rest[nbuf + 1 + nbuf:nbuf + 1 + 2 * nbuf]
        isem0, isem1 = rest[nbuf + 1 + 2 * nbuf:]
        c = lax.axis_index("c")
        s = lax.axis_index("s")
        yc = y_hbm.at[c]
        start = s * KBASE + jnp.minimum(s, KEXTRA)
        kc = jnp.where(s < KEXTRA, KMAX, KBASE)
        kfloor = (kc // nbuf) * nbuf

        @pl.when(s < KEXTRA)
        def _():
            pltpu.async_copy(src_hbm.at[pl.ds(start, KMAX)], src_v, isem0)
            pltpu.async_copy(dst_hbm.at[pl.ds(start, KMAX)], dst_v, isem1)

        @pl.when(s >= KEXTRA)
        def _():
            pltpu.async_copy(src_hbm.at[pl.ds(start, KBASE)],
                             src_v.at[pl.ds(0, KBASE)], isem0)
            pltpu.async_copy(dst_hbm.at[pl.ds(start, KBASE)],
                             dst_v.at[pl.ds(0, KBASE)], isem1)

        base = s * RPW
        _zero_fill(rows[0], dh)
        _init_accum(rows[0], accum, base)

        @pl.when(s < KEXTRA)
        def _():
            pltpu.make_async_copy(src_hbm.at[pl.ds(0, KMAX)], src_v,
                                  isem0).wait()
            pltpu.make_async_copy(dst_hbm.at[pl.ds(0, KMAX)], dst_v,
                                  isem1).wait()

        @pl.when(s >= KEXTRA)
        def _():
            pltpu.make_async_copy(src_hbm.at[pl.ds(0, KBASE)],
                                  src_v.at[pl.ds(0, KBASE)], isem0).wait()
            pltpu.make_async_copy(dst_hbm.at[pl.ds(0, KBASE)],
                                  dst_v.at[pl.ds(0, KBASE)], isem1).wait()
        plsc.subcore_barrier()

        for b in range(nbuf):
            pltpu.async_copy(yc.at[src_v.at[b]], rows[b], gsem[b])

        @pl.loop(0, kfloor, step=nbuf)
        def _(j):
            for b in range(nbuf):
                pltpu.make_async_copy(yc.at[src_v.at[0]], rows[b],
                                      gsem[b]).wait()
                pltpu.async_copy(rows[b], accum.at[dst_v.at[j + b]],
                                 ssem[b], add=True)
            for b in range(nbuf):
                pltpu.make_async_copy(rows[b], accum.at[dst_v.at[0]],
                                      ssem[b]).wait()

                @pl.when(j + b + nbuf < kc)
                def _():
                    pltpu.async_copy(yc.at[src_v.at[j + b + nbuf]],
                                     rows[b], gsem[b])

        for b in range(nbuf - 1):
            @pl.when(kfloor + b < kc)
            def _():
                pltpu.make_async_copy(yc.at[src_v.at[0]], rows[b],
                                      gsem[b]).wait()
                pltpu.async_copy(rows[b], accum.at[dst_v.at[kfloor + b]],
                                 ssem[b], add=True)
                pltpu.make_async_copy(rows[b], accum.at[dst_v.at[0]],
                                      ssem[b]).wait()

        plsc.subcore_barrier()
        pltpu.sync_copy(accum.at[pl.ds(base, RPW)],
                        out_hbm.at[c].at[pl.ds(base, RPW)])

    return sc_scatter


def _make_sc_hist():
    @functools.partial(
        pl.kernel,
        out_type=jax.ShapeDtypeStruct((NC, NPAD, HV), jnp.float32),
        mesh=plsc.VectorSubcoreMesh(**_MESH),
        compiler_params=pltpu.CompilerParams(use_tc_tiling_on_sc=False),
        scratch_types=[
            pltpu.VMEM((HMAX, 128), jnp.int32),
            pltpu.VMEM((128, HV), jnp.float32),
            pltpu.VMEM_SHARED((NPAD, HV), jnp.float32),
        ],
    )
    def sc_hist(dst_hbm, out_hbm, dst_v, vals, accum):
        c = lax.axis_index("c")
        s = lax.axis_index("s")
        w = c * NS + s
        start = w * HBASE + jnp.minimum(w, 2 * HEXTRA)
        kc = jnp.where(w < 2 * HEXTRA, HMAX, HBASE)

        @pl.when(w < 2 * HEXTRA)
        def _():
            pltpu.sync_copy(dst_hbm.at[pl.ds(start, HMAX)], dst_v)

        @pl.when(w >= 2 * HEXTRA)
        def _():
            pltpu.sync_copy(dst_hbm.at[pl.ds(start, HBASE)],
                            dst_v.at[pl.ds(0, HBASE)])

        base = s * RPW
        _zero_fill(vals, HV)
        _init_accum(vals, accum, base)
        plsc.subcore_barrier()

        ovec = jnp.ones((16,), jnp.float32)

        @pl.loop(0, 128)
        def _(r):
            vals[r, pl.ds(0, 16)] = ovec

        @pl.loop(0, kc)
        def _(j):
            pltpu.sync_copy(vals, accum.at[dst_v.at[j]], add=True)

        plsc.subcore_barrier()
        pltpu.sync_copy(accum.at[pl.ds(base, RPW)],
                        out_hbm.at[c].at[pl.ds(base, RPW)])

    return sc_hist


_SC_HIST = _make_sc_hist()
_SC_SCATTER = {d: _make_sc_scatter(d) for d in (128, 64, 32)}


def _node_spec(d):
    return pl.BlockSpec((NB, d), lambda i: (i, 0))


def _half_spec(dh):
    return pl.BlockSpec((2, NB, dh), lambda i: (0, i, 0))


def _full_spec(shape):
    return pl.BlockSpec(shape, lambda i: (0, 0))


def _dis(hp0_ref, hp1_ref):
    deg = hp0_ref[...][:, :1] + hp1_ref[...][:, :1] + 1.0
    return lax.rsqrt(deg)


def _dot(a, w_ref):
    return jnp.dot(a, w_ref[...], preferred_element_type=jnp.float32,
                   precision=lax.Precision.HIGHEST)


def _first_body(hp0_ref, hp1_ref, x_ref, w_ref, o_ref):
    y = _dis(hp0_ref, hp1_ref) * _dot(x_ref[...], w_ref)
    dh = o_ref.shape[2]
    o_ref[...] = jnp.stack([y[:, :dh], y[:, dh:]], axis=0)


def _combined(hp0_ref, hp1_ref, a_ref, y_ref, b_ref):
    a = a_ref[...] + y_ref[...]
    full = jnp.concatenate([a[0], a[1]], axis=1)
    p = _dis(hp0_ref, hp1_ref) * full
    return jnp.maximum(p + b_ref[...][:1, :], 0.0)


def _halves(y):
    dh = y.shape[1] // 2
    return jnp.stack([y[:, :dh], y[:, dh:]], axis=0)


def _make_combine_body(with_matmul):
    if with_matmul:
        def body(hp0_ref, hp1_ref, a_ref, y_ref, b_ref, w_ref, o_ref):
            h = _combined(hp0_ref, hp1_ref, a_ref, y_ref, b_ref)
            o_ref[...] = _halves(_dis(hp0_ref, hp1_ref) * _dot(h, w_ref))
    else:
        def body(hp0_ref, hp1_ref, a_ref, y_ref, b_ref, o_ref):
            h = _combined(hp0_ref, hp1_ref, a_ref, y_ref, b_ref)
            o_ref[...] = _halves(_dis(hp0_ref, hp1_ref) * h)
    return body


def _final_body(hp0_ref, hp1_ref, a_ref, y_ref, b_ref, w_ref, o_ref):
    a = a_ref[...] + y_ref[...]
    full = jnp.concatenate([a[0], a[1]], axis=1)
    p = _dis(hp0_ref, hp1_ref) * full
    o_ref[...] = _dot(p, w_ref) + b_ref[...][:1, :]


_HP_SPECS = [_node_spec(HV), _node_spec(HV)]


def _combine_specs(d, dn, with_w, db):
    dh = d // 2
    specs = _HP_SPECS + [_half_spec(dh), _half_spec(dh), _full_spec((8, db))]
    if with_w:
        specs.append(_full_spec((d, dn)))
    return specs


def _tc_call(body, in_specs, out_shape, out_spec, name):
    return pl.pallas_call(
        body,
        grid=(GRID,),
        in_specs=in_specs,
        out_specs=out_spec,
        out_shape=jax.ShapeDtypeStruct(out_shape, jnp.float32),
        name=name,
    )


def kernel(x, edge_index, W1, b1, W2, b2, W3, b3, W4, b4):
    xp = jnp.concatenate(
        [x, jnp.zeros((NPAD - N_NODES, D_FEAT), jnp.float32)], axis=0)
    ei = edge_index.astype(jnp.int32)
    src2 = ei[0].reshape(CHT, 128)
    dst2 = ei[1].reshape(CHT, 128)
    bb1 = jnp.broadcast_to(b1, (8, b1.shape[0]))
    bb2 = jnp.broadcast_to(b2, (8, b2.shape[0]))
    bb3 = jnp.broadcast_to(b3, (8, b3.shape[0]))
    bb4 = jnp.broadcast_to(b4, (8, b4.shape[0]))

    hp = _SC_HIST(dst2)
    hp0, hp1 = hp[0], hp[1]

    y1 = _tc_call(_first_body,
                  _HP_SPECS + [_node_spec(128), _full_spec((128, 128))],
                  (2, NPAD, 64), _half_spec(64),
                  "gcn_first")(hp0, hp1, xp, W1)
    g1 = _SC_SCATTER[128](y1, src2, dst2)
    y2 = _tc_call(_make_combine_body(True), _combine_specs(128, 64, True, 128),
                  (2, NPAD, 32), _half_spec(32),
                  "gcn_comb1")(hp0, hp1, g1, y1, bb1, W2)
    g2 = _SC_SCATTER[64](y2, src2, dst2)
    y3 = _tc_call(_make_combine_body(True), _combine_specs(64, 32, True, 64),
                  (2, NPAD, 16), _half_spec(16),
                  "gcn_comb2")(hp0, hp1, g2, y2, bb2, W3)
    g3 = _SC_SCATTER[32](y3, src2, dst2)
    y4 = _tc_call(_make_combine_body(False), _combine_specs(32, 0, False, 32),
                  (2, NPAD, 16), _half_spec(16),
                  "gcn_comb3")(hp0, hp1, g3, y3, bb3)
    g4 = _SC_SCATTER[32](y4, src2, dst2)
    out = _tc_call(_final_body, _combine_specs(32, 40, True, 40),
                   (NPAD, 40), _node_spec(40),
                   "gcn_final")(hp0, hp1, g4, y4, bb4, W4)
    return out[:N_NODES]
